# baseline (speedup 1.0000x reference)
import functools

import jax
import jax.numpy as jnp
from jax import lax
from jax.experimental import pallas as pl
from jax.experimental.pallas import tpu as pltpu
from jax.experimental.pallas import tpu_sc as plsc

B = 16384
K = 128
L = 16
NCH = 4


@functools.lru_cache(maxsize=None)
def _build(nc: int, ns: int):
    nw = nc * ns
    bpw = B // nw
    ch = bpw // NCH
    ch_vec = ch // L

    @functools.partial(
        pl.kernel,
        mesh=plsc.VectorSubcoreMesh(core_axis_name="c", subcore_axis_name="s"),
        out_type=jax.ShapeDtypeStruct((B,), jnp.float32),
        compiler_params=pltpu.CompilerParams(
            needs_layout_passes=False,
            skip_device_barrier=True,
            disable_bounds_checks=True,
            disable_semaphore_checks=True,
        ),
        scratch_types=[
            pltpu.VMEM((bpw,), jnp.float32),
            pltpu.VMEM((bpw, K), jnp.float32),
            pltpu.VMEM((bpw,), jnp.float32),
            pltpu.SemaphoreType.DMA,
            pltpu.SemaphoreType.DMA,
            pltpu.SemaphoreType.DMA,
            pltpu.SemaphoreType.DMA,
            pltpu.SemaphoreType.DMA,
        ],
    )
    def sc_gather(zt_hbm, a_hbm, out_hbm, zcol, av, vals, sem_z, *sem_a):
        wid = lax.axis_index("s") * nc + lax.axis_index("c")
        base = wid * bpw

        copies = [
            pltpu.async_copy(
                a_hbm.at[pl.ds(base + j * ch, ch)],
                av.at[pl.ds(j * ch, ch)],
                sem_a[j],
            )
            for j in range(NCH)
        ]
        cz = pltpu.async_copy(zt_hbm.at[0, pl.ds(base, bpw)], zcol, sem_z)
        cz.wait()

        for j in range(NCH):
            copies[j].wait()

            def step(i, carry):
                r16 = lax.iota(jnp.int32, L) + (i * L)
                slot16 = zcol[pl.ds(i * L, L)].astype(jnp.int32)
                v = plsc.load_gather(av, [r16, slot16])
                vals[pl.ds(i * L, L)] = v * jnp.float32(0.999)
                return carry

            lax.fori_loop(j * ch_vec, (j + 1) * ch_vec, step, 0)

        pltpu.sync_copy(vals, out_hbm.at[pl.ds(base, bpw)])

    return sc_gather


def kernel(z, a):
    info = plsc.get_sparse_core_info()
    return _build(info.num_cores, info.num_subcores)(z.T, a)


# speedup vs baseline: 1.0144x; 1.0144x over previous
import functools

import jax
import jax.numpy as jnp
from jax import lax
from jax.experimental import pallas as pl
from jax.experimental.pallas import tpu as pltpu
from jax.experimental.pallas import tpu_sc as plsc

B = 16384
K = 128
L = 16
NCH = 2


@functools.lru_cache(maxsize=None)
def _build(nc: int, ns: int):
    nw = nc * ns
    bpw = B // nw
    ch = bpw // NCH
    ch_vec = ch // L

    @functools.partial(
        pl.kernel,
        mesh=plsc.VectorSubcoreMesh(core_axis_name="c", subcore_axis_name="s"),
        out_type=jax.ShapeDtypeStruct((B,), jnp.float32),
        compiler_params=pltpu.CompilerParams(
            needs_layout_passes=False,
            skip_device_barrier=True,
            disable_bounds_checks=True,
            disable_semaphore_checks=True,
        ),
        scratch_types=[
            pltpu.VMEM((bpw,), jnp.float32),
            pltpu.VMEM((bpw, K), jnp.float32),
            pltpu.VMEM((bpw,), jnp.float32),
            pltpu.SemaphoreType.DMA,
            pltpu.SemaphoreType.DMA,
            pltpu.SemaphoreType.DMA,
            pltpu.SemaphoreType.DMA,
            pltpu.SemaphoreType.DMA,
        ],
    )
    def sc_gather(zt_hbm, a_hbm, out_hbm, zcol, av, vals, sem_z, *sem_a):
        wid = lax.axis_index("s") * nc + lax.axis_index("c")
        base = wid * bpw

        copies = [
            pltpu.async_copy(
                a_hbm.at[pl.ds(base + j * ch, ch)],
                av.at[pl.ds(j * ch, ch)],
                sem_a[j],
            )
            for j in range(NCH)
        ]
        cz = pltpu.async_copy(zt_hbm.at[0, pl.ds(base, bpw)], zcol, sem_z)
        cz.wait()

        for j in range(NCH):
            copies[j].wait()

            def step(i, carry):
                r16 = lax.iota(jnp.int32, L) + (i * L)
                slot16 = zcol[pl.ds(i * L, L)].astype(jnp.int32)
                v = plsc.load_gather(av, [r16, slot16])
                vals[pl.ds(i * L, L)] = v * jnp.float32(0.999)
                return carry

            lax.fori_loop(j * ch_vec, (j + 1) * ch_vec, step, 0)

        pltpu.sync_copy(vals, out_hbm.at[pl.ds(base, bpw)])

    return sc_gather


def kernel(z, a):
    info = plsc.get_sparse_core_info()
    return _build(info.num_cores, info.num_subcores)(z.T, a)
